# baseline (speedup 1.0000x reference)
import functools

import jax
import jax.numpy as jnp
import numpy as np
from jax import lax
from jax.experimental import pallas as pl
from jax.experimental.pallas import tpu as pltpu
from jax.experimental.pallas import tpu_sc as plsc

_EPS = 1e-05
_A1 = 0.3
_A2 = 0.3
_W0 = 1.0 - _A1 - _A2
_NUM_LABELS = 10

_KIN_DATA = np.array(
    [[64467757, 2916123636], [1705926158, 899080142],
     [1712723395, 2526649282], [2232176465, 33846082],
     [767915537, 735759787], [2252301940, 331845914],
     [2395792924, 649865367], [3515226245, 1150219387],
     [1308905690, 3242231867], [3647288517, 4265293960]], dtype=np.uint32)
_KOUT_DATA = np.array(
    [[2465931498, 255383827], [4095997477, 317277840],
     [91349104, 926951219], [2462096163, 4113027279],
     [3374067896, 3621954194], [1382268797, 2038861423],
     [3201614062, 502821546], [3650387604, 48903574],
     [272053746, 2003882608], [784671723, 584501553]], dtype=np.uint32)


def _style_src(labels):
    b = labels.shape[0]
    lab_vals = jnp.arange(_NUM_LABELS, dtype=labels.dtype)
    masks = labels[None, :] == lab_vals[:, None]
    counts = jnp.sum(masks, axis=1)
    present = counts > 0
    nbefore = jnp.cumsum(present) - present

    kin_keys = jax.random.wrap_key_data(jnp.asarray(_KIN_DATA)[nbefore])
    kout_keys = jax.random.wrap_key_data(jnp.asarray(_KOUT_DATA)[nbefore])

    js = jax.vmap(lambda k, m: jax.random.randint(k, (), 0, m))(
        kin_keys, counts - 1)
    jos = jax.vmap(lambda k, m: jax.random.randint(k, (), 0, m))(
        kout_keys, (b - counts) - 1)
    j_used = jnp.where(counts > 1, js, 0)

    rank_in = jnp.cumsum(masks, axis=1) - 1
    rank_out = jnp.cumsum(~masks, axis=1) - 1
    pick_in = jnp.argmax(masks & (rank_in == j_used[:, None]),
                         axis=1).astype(jnp.int32)
    pick_out = jnp.argmax((~masks) & (rank_out == jos[:, None]),
                          axis=1).astype(jnp.int32)

    src_in = pick_in[labels]
    src_out = pick_out[labels]
    return src_in, src_out


def _stats_body(x_ref, mean_ref, std_ref):
    x = x_ref[...]
    n = x.shape[-1]
    mean = jnp.mean(x, axis=-1)
    d = x - mean[:, :, None]
    var = jnp.sum(d * d, axis=-1) / (n - 1)
    mean_ref[...] = mean
    std_ref[...] = jnp.sqrt(var + _EPS)


_ROWS_PER_WORKER = 8
_LANES = 16


def _sc_combine_body(mean_hbm, std_hbm, si_hbm, so_hbm, scale_hbm, shift_hbm,
                     idx_in_v, idx_out_v, mb_v, sb_v, mi_v, sti_v, mo_v, sto_v,
                     scale_v, shift_v, sem):
    wid = lax.axis_index("s") * 2 + lax.axis_index("c")
    nw = 64 // _ROWS_PER_WORKER

    @pl.when(wid < nw)
    def _work():
        base = wid * _ROWS_PER_WORKER
        rows = pl.ds(base, _ROWS_PER_WORKER)
        pltpu.sync_copy(si_hbm, idx_in_v)
        pltpu.sync_copy(so_hbm, idx_out_v)
        pltpu.sync_copy(mean_hbm.at[rows], mb_v)
        pltpu.sync_copy(std_hbm.at[rows], sb_v)
        my_in = idx_in_v.at[rows]
        my_out = idx_out_v.at[rows]
        pltpu.async_copy(mean_hbm.at[my_in], mi_v, sem).wait()
        pltpu.async_copy(std_hbm.at[my_in], sti_v, sem).wait()
        pltpu.async_copy(mean_hbm.at[my_out], mo_v, sem).wait()
        pltpu.async_copy(std_hbm.at[my_out], sto_v, sem).wait()
        for r in range(_ROWS_PER_WORKER):
            for j in range(192 // _LANES):
                lsl = pl.ds(j * _LANES, _LANES)
                xm = mb_v[r, lsl]
                xs = sb_v[r, lsl]
                sc = (_W0 * xs + _A1 * sti_v[r, lsl]
                      + _A2 * sto_v[r, lsl]) / xs
                sh = (_W0 * xm + _A1 * mi_v[r, lsl]
                      + _A2 * mo_v[r, lsl]) - xm * sc
                scale_v[r, lsl] = sc
                shift_v[r, lsl] = sh
        pltpu.sync_copy(scale_v, scale_hbm.at[rows])
        pltpu.sync_copy(shift_v, shift_hbm.at[rows])


def _sc_combine(mean, std, src_in, src_out):
    b, c = mean.shape
    mesh = plsc.VectorSubcoreMesh(core_axis_name="c", subcore_axis_name="s",
                                  num_cores=2, num_subcores=16)
    f = pl.kernel(
        _sc_combine_body,
        out_type=[
            jax.ShapeDtypeStruct((b, c), jnp.float32),
            jax.ShapeDtypeStruct((b, c), jnp.float32),
        ],
        mesh=mesh,
        scratch_types=[
            pltpu.VMEM((b,), jnp.int32),
            pltpu.VMEM((b,), jnp.int32),
            pltpu.VMEM((_ROWS_PER_WORKER, c), jnp.float32),
            pltpu.VMEM((_ROWS_PER_WORKER, c), jnp.float32),
            pltpu.VMEM((_ROWS_PER_WORKER, c), jnp.float32),
            pltpu.VMEM((_ROWS_PER_WORKER, c), jnp.float32),
            pltpu.VMEM((_ROWS_PER_WORKER, c), jnp.float32),
            pltpu.VMEM((_ROWS_PER_WORKER, c), jnp.float32),
            pltpu.VMEM((_ROWS_PER_WORKER, c), jnp.float32),
            pltpu.VMEM((_ROWS_PER_WORKER, c), jnp.float32),
            pltpu.SemaphoreType.DMA,
        ],
    )
    return f(mean, std, src_in, src_out)


def _apply_body(x_ref, scale_ref, shift_ref, out_ref):
    s = scale_ref[...][:, :, None]
    t = shift_ref[...][:, :, None]
    out_ref[...] = x_ref[...] * s + t


def kernel(content, labels):
    b, c, h, w = content.shape
    hw = h * w
    x = content.reshape(b, c, hw)
    src_in, src_out = _style_src(labels)

    bb = 8
    mean, std = pl.pallas_call(
        _stats_body,
        grid=(b // bb,),
        in_specs=[pl.BlockSpec((bb, c, hw), lambda i: (i, 0, 0))],
        out_specs=[
            pl.BlockSpec((bb, c), lambda i: (i, 0)),
            pl.BlockSpec((bb, c), lambda i: (i, 0)),
        ],
        out_shape=[
            jax.ShapeDtypeStruct((b, c), jnp.float32),
            jax.ShapeDtypeStruct((b, c), jnp.float32),
        ],
    )(x)

    mean_p = jnp.pad(mean, ((0, 0), (0, 64)))
    std_p = jnp.pad(std, ((0, 0), (0, 64)), constant_values=1.0)
    scale_p, shift_p = _sc_combine(mean_p, std_p, src_in, src_out)
    scale = scale_p[:, :c]
    shift = shift_p[:, :c]

    out = pl.pallas_call(
        _apply_body,
        grid=(b // bb,),
        in_specs=[
            pl.BlockSpec((bb, c, hw), lambda i: (i, 0, 0)),
            pl.BlockSpec((bb, c), lambda i: (i, 0)),
            pl.BlockSpec((bb, c), lambda i: (i, 0)),
        ],
        out_specs=pl.BlockSpec((bb, c, hw), lambda i: (i, 0, 0)),
        out_shape=jax.ShapeDtypeStruct((b, c, hw), jnp.float32),
    )(x, scale, shift)
    return out.reshape(b, c, h, w)
